# baseline (speedup 1.0000x reference)
import functools

import jax
import jax.numpy as jnp
from jax import lax
from jax.experimental import pallas as pl
from jax.experimental.pallas import tpu as pltpu
from jax.experimental.pallas import tpu_sc as plsc

NUM_BINS = 5
BATCH = 16384

_HALF_LOG_2PI = 0.9189385332046727
_CF_ITERS = 8
_FPMIN = 1e-30


_NC = 2
_NS = 16
_NW = _NC * _NS
_BPW = BATCH // _NW


def _sc_gather_body(uid_hbm, iid_hbm, ut_hbm, it_hbm, out_hbm,
                    uidx_v, iidx_v, urows_v, irows_v, sem_u, sem_i):
    wid = lax.axis_index("s") * _NC + lax.axis_index("c")
    base = wid * _BPW
    pltpu.sync_copy(uid_hbm.at[pl.ds(base, _BPW)], uidx_v)
    pltpu.sync_copy(iid_hbm.at[pl.ds(base, _BPW)], iidx_v)
    cps = []
    for k in range(NUM_BINS):
        cps.append(pltpu.async_copy(ut_hbm.at[k].at[uidx_v],
                                    urows_v.at[pl.ds(k * _BPW, _BPW)], sem_u))
        cps.append(pltpu.async_copy(it_hbm.at[k].at[iidx_v],
                                    irows_v.at[pl.ds(k * _BPW, _BPW)], sem_i))
    for cp in cps:
        cp.wait()

    def add_chunk(j, _):
        sl = pl.ds(j * 16, 16)
        urows_v[sl] = urows_v[sl] + irows_v[sl]
        return _
    lax.fori_loop(0, NUM_BINS * _BPW // 16, add_chunk, 0)
    for k in range(NUM_BINS):
        pltpu.sync_copy(urows_v.at[pl.ds(k * _BPW, _BPW)],
                        out_hbm.at[k, pl.ds(base, _BPW)])


def _sc_gather(uid, iid, ut_t, it_t):
    mesh = plsc.VectorSubcoreMesh(core_axis_name="c", subcore_axis_name="s")
    f = pl.kernel(
        _sc_gather_body,
        mesh=mesh,
        compiler_params=pltpu.CompilerParams(use_tc_tiling_on_sc=False),
        out_type=jax.ShapeDtypeStruct((NUM_BINS, BATCH), jnp.float32),
        scratch_types=[
            pltpu.VMEM((_BPW,), jnp.int32),
            pltpu.VMEM((_BPW,), jnp.int32),
            pltpu.VMEM((NUM_BINS * _BPW,), jnp.float32),
            pltpu.VMEM((NUM_BINS * _BPW,), jnp.float32),
            pltpu.SemaphoreType.DMA,
            pltpu.SemaphoreType.DMA,
        ],
    )
    return f(uid, iid, ut_t, it_t)


def _lgamma_small(x):
    z = x + 6.0
    shift = (jnp.log(x) + jnp.log(x + 1.0) + jnp.log(x + 2.0)
             + jnp.log(x + 3.0) + jnp.log(x + 4.0) + jnp.log(x + 5.0))
    zi = 1.0 / z
    zi2 = zi * zi
    st = (z - 0.5) * jnp.log(z) - z + _HALF_LOG_2PI + zi * (
        1.0 / 12.0 - zi2 * (1.0 / 360.0))
    return st - shift


def _betainc_cf(a, b, x):
    thresh = (a + 1.0) / (a + b + 2.0)
    swap = x > thresh
    aa = jnp.where(swap, b, a)
    bb = jnp.where(swap, a, b)
    xx = jnp.where(swap, 1.0 - x, x)
    xx = jnp.clip(xx, _FPMIN, 1.0)
    lnpre = (aa * jnp.log(xx) + bb * jnp.log(1.0 - xx)
             + _lgamma_small(aa + bb) - _lgamma_small(aa) - _lgamma_small(bb))
    front = jnp.exp(lnpre) / aa
    qab = aa + bb
    qap = aa + 1.0
    qam = aa - 1.0
    c = jnp.ones_like(xx)
    d = 1.0 - qab * xx / qap
    d = jnp.where(jnp.abs(d) < _FPMIN, _FPMIN, d)
    d = 1.0 / d
    h = d
    for m in range(1, _CF_ITERS + 1):
        m2 = 2.0 * m
        num = m * (bb - m) * xx / ((qam + m2) * (aa + m2))
        d = 1.0 + num * d
        d = jnp.where(jnp.abs(d) < _FPMIN, _FPMIN, d)
        c = 1.0 + num / c
        c = jnp.where(jnp.abs(c) < _FPMIN, _FPMIN, c)
        d = 1.0 / d
        h = h * d * c
        num = -(aa + m) * (qab + m) * xx / ((aa + m2) * (qap + m2))
        d = 1.0 + num * d
        d = jnp.where(jnp.abs(d) < _FPMIN, _FPMIN, d)
        c = 1.0 + num / c
        c = jnp.where(jnp.abs(c) < _FPMIN, _FPMIN, c)
        d = 1.0 / d
        h = h * d * c
    res = front * h
    return jnp.where(swap, 1.0 - res, res)


def _tc_body(s_ref, a_ref, b_ref, mass_ref, edges_ref):
    e = jnp.exp(s_ref[...])
    r0 = e[0:1, :]
    r1 = r0 + e[1:2, :]
    r2 = r1 + e[2:3, :]
    r3 = r2 + e[3:4, :]
    tot = r3 + e[4:5, :]
    inv = 1.0 / tot
    e0 = r0 * inv
    e1 = r1 * inv
    e2 = r2 * inv
    e3 = r3 * inv
    ones = jnp.ones_like(e0)
    edges_ref[...] = jnp.concatenate([e0, e1, e2, e3, ones], axis=0)
    x4 = jnp.concatenate([e0, e1, e2, e3], axis=0)
    a4 = jnp.broadcast_to(a_ref[...], x4.shape)
    b4 = jnp.broadcast_to(b_ref[...], x4.shape)
    cdf = _betainc_cf(a4, b4, x4)
    c0 = cdf[0:1, :]
    c1 = cdf[1:2, :]
    c2 = cdf[2:3, :]
    c3 = cdf[3:4, :]
    mass_ref[...] = jnp.concatenate(
        [c0, c1 - c0, c2 - c1, c3 - c2, ones - c3], axis=0)


def _tc_math(s_t, a_t, b_t, block_l=2048):
    nblk = BATCH // block_l
    spec5 = pl.BlockSpec((NUM_BINS, block_l), lambda j: (0, j))
    spec1 = pl.BlockSpec((1, block_l), lambda j: (0, j))
    return pl.pallas_call(
        _tc_body,
        grid=(nblk,),
        in_specs=[spec5, spec1, spec1],
        out_specs=[spec5, spec5],
        out_shape=(
            jax.ShapeDtypeStruct((NUM_BINS, BATCH), jnp.float32),
            jax.ShapeDtypeStruct((NUM_BINS, BATCH), jnp.float32),
        ),
    )(s_t, a_t, b_t)


def kernel(uid, iid, alpha, beta, uid_table, iid_table):
    s_t = _sc_gather(uid.astype(jnp.int32), iid.astype(jnp.int32),
                     uid_table.T, iid_table.T)
    a_t = alpha.reshape(1, BATCH)
    b_t = beta.reshape(1, BATCH)
    mass_t, edges_t = _tc_math(s_t, a_t, b_t)
    return (mass_t.T, edges_t.T)


# speedup vs baseline: 4.2537x; 4.2537x over previous
import functools

import jax
import jax.numpy as jnp
from jax import lax
from jax.experimental import pallas as pl
from jax.experimental.pallas import tpu as pltpu
from jax.experimental.pallas import tpu_sc as plsc

NUM_BINS = 5
BATCH = 16384

_HALF_LOG_2PI = 0.9189385332046727
_CF_ITERS = 8
_FPMIN = 1e-30


_NC = 2
_NS = 16
_NW = _NC * _NS
_BPW = BATCH // _NW


def _sc_gather_body(uid_hbm, iid_hbm, u0, u1, u2, u3, u4, i0, i1, i2, i3, i4,
                    out_hbm, uidx_v, iidx_v, urows_v, irows_v, sem_u, sem_i):
    ut_bins = (u0, u1, u2, u3, u4)
    it_bins = (i0, i1, i2, i3, i4)
    wid = lax.axis_index("s") * _NC + lax.axis_index("c")
    base = wid * _BPW
    pltpu.sync_copy(uid_hbm.at[pl.ds(base, _BPW)], uidx_v)
    pltpu.sync_copy(iid_hbm.at[pl.ds(base, _BPW)], iidx_v)
    cps = []
    for k in range(NUM_BINS):
        cps.append(pltpu.async_copy(ut_bins[k].at[uidx_v],
                                    urows_v.at[pl.ds(k * _BPW, _BPW)], sem_u))
        cps.append(pltpu.async_copy(it_bins[k].at[iidx_v],
                                    irows_v.at[pl.ds(k * _BPW, _BPW)], sem_i))
    for cp in cps:
        cp.wait()

    def add_chunk(j, _):
        sl = pl.ds(j * 16, 16)
        urows_v[sl] = urows_v[sl] + irows_v[sl]
        return _
    lax.fori_loop(0, NUM_BINS * _BPW // 16, add_chunk, 0)
    for k in range(NUM_BINS):
        pltpu.sync_copy(urows_v.at[pl.ds(k * _BPW, _BPW)],
                        out_hbm.at[k, pl.ds(base, _BPW)])


def _sc_gather(uid, iid, u_bins, i_bins):
    mesh = plsc.VectorSubcoreMesh(core_axis_name="c", subcore_axis_name="s")
    f = pl.kernel(
        _sc_gather_body,
        mesh=mesh,
        compiler_params=pltpu.CompilerParams(use_tc_tiling_on_sc=False),
        out_type=jax.ShapeDtypeStruct((NUM_BINS, BATCH), jnp.float32),
        scratch_types=[
            pltpu.VMEM((_BPW,), jnp.int32),
            pltpu.VMEM((_BPW,), jnp.int32),
            pltpu.VMEM((NUM_BINS * _BPW,), jnp.float32),
            pltpu.VMEM((NUM_BINS * _BPW,), jnp.float32),
            pltpu.SemaphoreType.DMA,
            pltpu.SemaphoreType.DMA,
        ],
    )
    return f(uid, iid, *u_bins, *i_bins)


def _lgamma_small(x):
    z = x + 6.0
    shift = (jnp.log(x) + jnp.log(x + 1.0) + jnp.log(x + 2.0)
             + jnp.log(x + 3.0) + jnp.log(x + 4.0) + jnp.log(x + 5.0))
    zi = 1.0 / z
    zi2 = zi * zi
    st = (z - 0.5) * jnp.log(z) - z + _HALF_LOG_2PI + zi * (
        1.0 / 12.0 - zi2 * (1.0 / 360.0))
    return st - shift


def _betainc_cf(a, b, x):
    thresh = (a + 1.0) / (a + b + 2.0)
    swap = x > thresh
    aa = jnp.where(swap, b, a)
    bb = jnp.where(swap, a, b)
    xx = jnp.where(swap, 1.0 - x, x)
    xx = jnp.clip(xx, _FPMIN, 1.0)
    lnpre = (aa * jnp.log(xx) + bb * jnp.log(1.0 - xx)
             + _lgamma_small(aa + bb) - _lgamma_small(aa) - _lgamma_small(bb))
    front = jnp.exp(lnpre) / aa
    qab = aa + bb
    qap = aa + 1.0
    qam = aa - 1.0
    c = jnp.ones_like(xx)
    d = 1.0 - qab * xx / qap
    d = jnp.where(jnp.abs(d) < _FPMIN, _FPMIN, d)
    d = 1.0 / d
    h = d
    for m in range(1, _CF_ITERS + 1):
        m2 = 2.0 * m
        num = m * (bb - m) * xx / ((qam + m2) * (aa + m2))
        d = 1.0 + num * d
        d = jnp.where(jnp.abs(d) < _FPMIN, _FPMIN, d)
        c = 1.0 + num / c
        c = jnp.where(jnp.abs(c) < _FPMIN, _FPMIN, c)
        d = 1.0 / d
        h = h * d * c
        num = -(aa + m) * (qab + m) * xx / ((aa + m2) * (qap + m2))
        d = 1.0 + num * d
        d = jnp.where(jnp.abs(d) < _FPMIN, _FPMIN, d)
        c = 1.0 + num / c
        c = jnp.where(jnp.abs(c) < _FPMIN, _FPMIN, c)
        d = 1.0 / d
        h = h * d * c
    res = front * h
    return jnp.where(swap, 1.0 - res, res)


def _tc_body(s_ref, a_ref, b_ref, mass_ref, edges_ref):
    e = jnp.exp(s_ref[...])
    r0 = e[0:1, :]
    r1 = r0 + e[1:2, :]
    r2 = r1 + e[2:3, :]
    r3 = r2 + e[3:4, :]
    tot = r3 + e[4:5, :]
    inv = 1.0 / tot
    e0 = r0 * inv
    e1 = r1 * inv
    e2 = r2 * inv
    e3 = r3 * inv
    ones = jnp.ones_like(e0)
    edges_ref[...] = jnp.concatenate([e0, e1, e2, e3, ones], axis=0)
    x4 = jnp.concatenate([e0, e1, e2, e3], axis=0)
    a4 = jnp.broadcast_to(a_ref[...], x4.shape)
    b4 = jnp.broadcast_to(b_ref[...], x4.shape)
    cdf = _betainc_cf(a4, b4, x4)
    c0 = cdf[0:1, :]
    c1 = cdf[1:2, :]
    c2 = cdf[2:3, :]
    c3 = cdf[3:4, :]
    mass_ref[...] = jnp.concatenate(
        [c0, c1 - c0, c2 - c1, c3 - c2, ones - c3], axis=0)


def _tc_math(s_t, a_t, b_t, block_l=2048):
    nblk = BATCH // block_l
    spec5 = pl.BlockSpec((NUM_BINS, block_l), lambda j: (0, j))
    spec1 = pl.BlockSpec((1, block_l), lambda j: (0, j))
    return pl.pallas_call(
        _tc_body,
        grid=(nblk,),
        in_specs=[spec5, spec1, spec1],
        out_specs=[spec5, spec5],
        out_shape=(
            jax.ShapeDtypeStruct((NUM_BINS, BATCH), jnp.float32),
            jax.ShapeDtypeStruct((NUM_BINS, BATCH), jnp.float32),
        ),
    )(s_t, a_t, b_t)


def kernel(uid, iid, alpha, beta, uid_table, iid_table):
    ut_t = uid_table.T
    it_t = iid_table.T
    u_bins = [ut_t[k] for k in range(NUM_BINS)]
    i_bins = [it_t[k] for k in range(NUM_BINS)]
    s_t = _sc_gather(uid.astype(jnp.int32), iid.astype(jnp.int32),
                     u_bins, i_bins)
    a_t = alpha.reshape(1, BATCH)
    b_t = beta.reshape(1, BATCH)
    mass_t, edges_t = _tc_math(s_t, a_t, b_t)
    return (mass_t.T, edges_t.T)


# speedup vs baseline: 8.3632x; 1.9661x over previous
import functools

import jax
import jax.numpy as jnp
from jax import lax
from jax.experimental import pallas as pl
from jax.experimental.pallas import tpu as pltpu
from jax.experimental.pallas import tpu_sc as plsc

NUM_BINS = 5
BATCH = 16384

_HALF_LOG_2PI = 0.9189385332046727
_CF_ITERS = 8
_FPMIN = 1e-30


_NC = 2
_NS = 16
_NW = _NC * _NS
_BPW = BATCH // _NW


def _sc_gather_body(uid_hbm, iid_hbm, u0, u1, u2, u3, u4, i0, i1, i2, i3, i4,
                    out_hbm, uidx_v, iidx_v, urows_v, irows_v, sem_u, sem_i):
    ut_bins = (u0, u1, u2, u3, u4)
    it_bins = (i0, i1, i2, i3, i4)
    wid = lax.axis_index("s") * _NC + lax.axis_index("c")
    base = wid * _BPW
    pltpu.sync_copy(uid_hbm.at[pl.ds(base, _BPW)], uidx_v)
    pltpu.sync_copy(iid_hbm.at[pl.ds(base, _BPW)], iidx_v)
    cps = []
    for k in range(NUM_BINS):
        cps.append(pltpu.async_copy(ut_bins[k].at[uidx_v],
                                    urows_v.at[pl.ds(k * _BPW, _BPW)], sem_u))
        cps.append(pltpu.async_copy(it_bins[k].at[iidx_v],
                                    irows_v.at[pl.ds(k * _BPW, _BPW)], sem_i))
    for cp in cps:
        cp.wait()

    def add_chunk(j, _):
        sl = pl.ds(j * 16, 16)
        urows_v[sl] = urows_v[sl] + irows_v[sl]
        return _
    lax.fori_loop(0, NUM_BINS * _BPW // 16, add_chunk, 0)
    for k in range(NUM_BINS):
        pltpu.sync_copy(urows_v.at[pl.ds(k * _BPW, _BPW)],
                        out_hbm.at[k, pl.ds(base, _BPW)])


def _sc_gather(uid, iid, u_bins, i_bins):
    mesh = plsc.VectorSubcoreMesh(core_axis_name="c", subcore_axis_name="s")
    f = pl.kernel(
        _sc_gather_body,
        mesh=mesh,
        compiler_params=pltpu.CompilerParams(use_tc_tiling_on_sc=False),
        out_type=jax.ShapeDtypeStruct((NUM_BINS, BATCH), jnp.float32),
        scratch_types=[
            pltpu.VMEM((_BPW,), jnp.int32),
            pltpu.VMEM((_BPW,), jnp.int32),
            pltpu.VMEM((NUM_BINS * _BPW,), jnp.float32),
            pltpu.VMEM((NUM_BINS * _BPW,), jnp.float32),
            pltpu.SemaphoreType.DMA,
            pltpu.SemaphoreType.DMA,
        ],
    )
    return f(uid, iid, *u_bins, *i_bins)


_NROWS = 1000000
_DETILE_BLK = 16384


def _tc_detile_body(u_ref, i_ref, *out_refs):
    for k in range(NUM_BINS):
        out_refs[k][...] = u_ref[k, :]
        out_refs[NUM_BINS + k][...] = i_ref[k, :]


def _tc_detile(ut_t, it_t):
    nblk = pl.cdiv(_NROWS, _DETILE_BLK)
    spec_in = pl.BlockSpec((NUM_BINS, _DETILE_BLK), lambda j: (0, j))
    spec_out = pl.BlockSpec((_DETILE_BLK,), lambda j: (j,))
    return pl.pallas_call(
        _tc_detile_body,
        grid=(nblk,),
        in_specs=[spec_in, spec_in],
        out_specs=[spec_out] * (2 * NUM_BINS),
        out_shape=tuple(jax.ShapeDtypeStruct((_NROWS,), jnp.float32)
                        for _ in range(2 * NUM_BINS)),
    )(ut_t, it_t)


def _lgamma_small(x):
    z = x + 6.0
    shift = (jnp.log(x) + jnp.log(x + 1.0) + jnp.log(x + 2.0)
             + jnp.log(x + 3.0) + jnp.log(x + 4.0) + jnp.log(x + 5.0))
    zi = 1.0 / z
    zi2 = zi * zi
    st = (z - 0.5) * jnp.log(z) - z + _HALF_LOG_2PI + zi * (
        1.0 / 12.0 - zi2 * (1.0 / 360.0))
    return st - shift


def _betainc_cf(a, b, x):
    thresh = (a + 1.0) / (a + b + 2.0)
    swap = x > thresh
    aa = jnp.where(swap, b, a)
    bb = jnp.where(swap, a, b)
    xx = jnp.where(swap, 1.0 - x, x)
    xx = jnp.clip(xx, _FPMIN, 1.0)
    lnpre = (aa * jnp.log(xx) + bb * jnp.log(1.0 - xx)
             + _lgamma_small(aa + bb) - _lgamma_small(aa) - _lgamma_small(bb))
    front = jnp.exp(lnpre) / aa
    qab = aa + bb
    qap = aa + 1.0
    qam = aa - 1.0
    c = jnp.ones_like(xx)
    d = 1.0 - qab * xx / qap
    d = jnp.where(jnp.abs(d) < _FPMIN, _FPMIN, d)
    d = 1.0 / d
    h = d
    for m in range(1, _CF_ITERS + 1):
        m2 = 2.0 * m
        num = m * (bb - m) * xx / ((qam + m2) * (aa + m2))
        d = 1.0 + num * d
        d = jnp.where(jnp.abs(d) < _FPMIN, _FPMIN, d)
        c = 1.0 + num / c
        c = jnp.where(jnp.abs(c) < _FPMIN, _FPMIN, c)
        d = 1.0 / d
        h = h * d * c
        num = -(aa + m) * (qab + m) * xx / ((aa + m2) * (qap + m2))
        d = 1.0 + num * d
        d = jnp.where(jnp.abs(d) < _FPMIN, _FPMIN, d)
        c = 1.0 + num / c
        c = jnp.where(jnp.abs(c) < _FPMIN, _FPMIN, c)
        d = 1.0 / d
        h = h * d * c
    res = front * h
    return jnp.where(swap, 1.0 - res, res)


def _tc_body(s_ref, a_ref, b_ref, mass_ref, edges_ref):
    e = jnp.exp(s_ref[...])
    r0 = e[0:1, :]
    r1 = r0 + e[1:2, :]
    r2 = r1 + e[2:3, :]
    r3 = r2 + e[3:4, :]
    tot = r3 + e[4:5, :]
    inv = 1.0 / tot
    e0 = r0 * inv
    e1 = r1 * inv
    e2 = r2 * inv
    e3 = r3 * inv
    ones = jnp.ones_like(e0)
    edges_ref[...] = jnp.concatenate([e0, e1, e2, e3, ones], axis=0)
    x4 = jnp.concatenate([e0, e1, e2, e3], axis=0)
    a4 = jnp.broadcast_to(a_ref[...], x4.shape)
    b4 = jnp.broadcast_to(b_ref[...], x4.shape)
    cdf = _betainc_cf(a4, b4, x4)
    c0 = cdf[0:1, :]
    c1 = cdf[1:2, :]
    c2 = cdf[2:3, :]
    c3 = cdf[3:4, :]
    mass_ref[...] = jnp.concatenate(
        [c0, c1 - c0, c2 - c1, c3 - c2, ones - c3], axis=0)


def _tc_math(s_t, a_t, b_t, block_l=2048):
    nblk = BATCH // block_l
    spec5 = pl.BlockSpec((NUM_BINS, block_l), lambda j: (0, j))
    spec1 = pl.BlockSpec((1, block_l), lambda j: (0, j))
    return pl.pallas_call(
        _tc_body,
        grid=(nblk,),
        in_specs=[spec5, spec1, spec1],
        out_specs=[spec5, spec5],
        out_shape=(
            jax.ShapeDtypeStruct((NUM_BINS, BATCH), jnp.float32),
            jax.ShapeDtypeStruct((NUM_BINS, BATCH), jnp.float32),
        ),
    )(s_t, a_t, b_t)


def kernel(uid, iid, alpha, beta, uid_table, iid_table):
    ut_t = uid_table.T
    it_t = iid_table.T
    bins = _tc_detile(ut_t, it_t)
    u_bins = bins[:NUM_BINS]
    i_bins = bins[NUM_BINS:]
    s_t = _sc_gather(uid.astype(jnp.int32), iid.astype(jnp.int32),
                     u_bins, i_bins)
    a_t = alpha.reshape(1, BATCH)
    b_t = beta.reshape(1, BATCH)
    mass_t, edges_t = _tc_math(s_t, a_t, b_t)
    return (mass_t.T, edges_t.T)


# speedup vs baseline: 11.2139x; 1.3409x over previous
import functools

import jax
import jax.numpy as jnp
from jax import lax
from jax.experimental import pallas as pl
from jax.experimental.pallas import tpu as pltpu
from jax.experimental.pallas import tpu_sc as plsc

NUM_BINS = 5
BATCH = 16384

_HALF_LOG_2PI = 0.9189385332046727
_CF_ITERS = 8
_FPMIN = 1e-30


_NC = 2
_NS = 16
_NW = _NC * _NS
_BPW = BATCH // _NW


def _sc_gather_body(uid_hbm, iid_hbm, u0, u1, u2, u3, u4, i0, i1, i2, i3, i4,
                    out_hbm, uidx_v, iidx_v, urows_v, irows_v, sem_u, sem_i):
    ut_bins = (u0, u1, u2, u3, u4)
    it_bins = (i0, i1, i2, i3, i4)
    wid = lax.axis_index("s") * _NC + lax.axis_index("c")
    base = wid * _BPW
    pltpu.sync_copy(uid_hbm.at[pl.ds(base, _BPW)], uidx_v)
    pltpu.sync_copy(iid_hbm.at[pl.ds(base, _BPW)], iidx_v)
    cps = []
    for k in range(NUM_BINS):
        cps.append(pltpu.async_copy(ut_bins[k].at[uidx_v],
                                    urows_v.at[pl.ds(k * _BPW, _BPW)], sem_u))
        cps.append(pltpu.async_copy(it_bins[k].at[iidx_v],
                                    irows_v.at[pl.ds(k * _BPW, _BPW)], sem_i))
    for cp in cps:
        cp.wait()

    def add_chunk(j, _):
        sl = pl.ds(j * 16, 16)
        urows_v[sl] = urows_v[sl] + irows_v[sl]
        return _
    lax.fori_loop(0, NUM_BINS * _BPW // 16, add_chunk, 0)
    for k in range(NUM_BINS):
        pltpu.sync_copy(urows_v.at[pl.ds(k * _BPW, _BPW)],
                        out_hbm.at[k, pl.ds(base, _BPW)])


def _sc_gather(uid, iid, u_bins, i_bins):
    mesh = plsc.VectorSubcoreMesh(core_axis_name="c", subcore_axis_name="s")
    f = pl.kernel(
        _sc_gather_body,
        mesh=mesh,
        compiler_params=pltpu.CompilerParams(use_tc_tiling_on_sc=False),
        out_type=jax.ShapeDtypeStruct((NUM_BINS, BATCH), jnp.float32),
        scratch_types=[
            pltpu.VMEM((_BPW,), jnp.int32),
            pltpu.VMEM((_BPW,), jnp.int32),
            pltpu.VMEM((NUM_BINS * _BPW,), jnp.float32),
            pltpu.VMEM((NUM_BINS * _BPW,), jnp.float32),
            pltpu.SemaphoreType.DMA,
            pltpu.SemaphoreType.DMA,
        ],
    )
    return f(uid, iid, *u_bins, *i_bins)


_NROWS = 1000000


_DETILE_BLK = 65536


def _tc_detile_body(u_ref, i_ref, *out_refs):
    for k in range(NUM_BINS):
        out_refs[k][...] = u_ref[k, :]
        out_refs[NUM_BINS + k][...] = i_ref[k, :]


def _tc_detile(ut_t, it_t):
    nblk = pl.cdiv(_NROWS, _DETILE_BLK)
    spec_in = pl.BlockSpec((NUM_BINS, _DETILE_BLK), lambda j: (0, j))
    spec_out = pl.BlockSpec((_DETILE_BLK,), lambda j: (j,))
    return pl.pallas_call(
        _tc_detile_body,
        grid=(nblk,),
        in_specs=[spec_in, spec_in],
        out_specs=[spec_out] * (2 * NUM_BINS),
        out_shape=tuple(jax.ShapeDtypeStruct((_NROWS,), jnp.float32)
                        for _ in range(2 * NUM_BINS)),
    )(ut_t, it_t)


def _lgamma_small(x):
    z = x + 6.0
    shift = (jnp.log(x) + jnp.log(x + 1.0) + jnp.log(x + 2.0)
             + jnp.log(x + 3.0) + jnp.log(x + 4.0) + jnp.log(x + 5.0))
    zi = 1.0 / z
    zi2 = zi * zi
    st = (z - 0.5) * jnp.log(z) - z + _HALF_LOG_2PI + zi * (
        1.0 / 12.0 - zi2 * (1.0 / 360.0))
    return st - shift


def _betainc_cf(a, b, x):
    thresh = (a + 1.0) / (a + b + 2.0)
    swap = x > thresh
    aa = jnp.where(swap, b, a)
    bb = jnp.where(swap, a, b)
    xx = jnp.where(swap, 1.0 - x, x)
    xx = jnp.clip(xx, _FPMIN, 1.0)
    lnpre = (aa * jnp.log(xx) + bb * jnp.log(1.0 - xx)
             + _lgamma_small(aa + bb) - _lgamma_small(aa) - _lgamma_small(bb))
    front = jnp.exp(lnpre) / aa
    qab = aa + bb
    qap = aa + 1.0
    qam = aa - 1.0
    c = jnp.ones_like(xx)
    d = 1.0 - qab * xx / qap
    d = jnp.where(jnp.abs(d) < _FPMIN, _FPMIN, d)
    d = 1.0 / d
    h = d
    for m in range(1, _CF_ITERS + 1):
        m2 = 2.0 * m
        num = m * (bb - m) * xx / ((qam + m2) * (aa + m2))
        d = 1.0 + num * d
        d = jnp.where(jnp.abs(d) < _FPMIN, _FPMIN, d)
        c = 1.0 + num / c
        c = jnp.where(jnp.abs(c) < _FPMIN, _FPMIN, c)
        d = 1.0 / d
        h = h * d * c
        num = -(aa + m) * (qab + m) * xx / ((aa + m2) * (qap + m2))
        d = 1.0 + num * d
        d = jnp.where(jnp.abs(d) < _FPMIN, _FPMIN, d)
        c = 1.0 + num / c
        c = jnp.where(jnp.abs(c) < _FPMIN, _FPMIN, c)
        d = 1.0 / d
        h = h * d * c
    res = front * h
    return jnp.where(swap, 1.0 - res, res)


def _tc_body(s_ref, a_ref, b_ref, mass_ref, edges_ref):
    e = jnp.exp(s_ref[...])
    r0 = e[0:1, :]
    r1 = r0 + e[1:2, :]
    r2 = r1 + e[2:3, :]
    r3 = r2 + e[3:4, :]
    tot = r3 + e[4:5, :]
    inv = 1.0 / tot
    e0 = r0 * inv
    e1 = r1 * inv
    e2 = r2 * inv
    e3 = r3 * inv
    ones = jnp.ones_like(e0)
    edges_ref[...] = jnp.concatenate([e0, e1, e2, e3, ones], axis=0)
    x4 = jnp.concatenate([e0, e1, e2, e3], axis=0)
    a4 = jnp.broadcast_to(a_ref[...], x4.shape)
    b4 = jnp.broadcast_to(b_ref[...], x4.shape)
    cdf = _betainc_cf(a4, b4, x4)
    c0 = cdf[0:1, :]
    c1 = cdf[1:2, :]
    c2 = cdf[2:3, :]
    c3 = cdf[3:4, :]
    mass_ref[...] = jnp.concatenate(
        [c0, c1 - c0, c2 - c1, c3 - c2, ones - c3], axis=0)


def _tc_math(s_t, a_t, b_t, block_l=2048):
    nblk = BATCH // block_l
    spec5 = pl.BlockSpec((NUM_BINS, block_l), lambda j: (0, j))
    spec1 = pl.BlockSpec((1, block_l), lambda j: (0, j))
    return pl.pallas_call(
        _tc_body,
        grid=(nblk,),
        in_specs=[spec5, spec1, spec1],
        out_specs=[spec5, spec5],
        out_shape=(
            jax.ShapeDtypeStruct((NUM_BINS, BATCH), jnp.float32),
            jax.ShapeDtypeStruct((NUM_BINS, BATCH), jnp.float32),
        ),
    )(s_t, a_t, b_t)


def kernel(uid, iid, alpha, beta, uid_table, iid_table):
    ut_t = uid_table.T
    it_t = iid_table.T
    bins = _tc_detile(ut_t, it_t)
    u_bins = bins[:NUM_BINS]
    i_bins = bins[NUM_BINS:]
    s_t = _sc_gather(uid.astype(jnp.int32), iid.astype(jnp.int32),
                     u_bins, i_bins)
    a_t = alpha.reshape(1, BATCH)
    b_t = beta.reshape(1, BATCH)
    mass_t, edges_t = _tc_math(s_t, a_t, b_t)
    return (mass_t.T, edges_t.T)


# speedup vs baseline: 11.4389x; 1.0201x over previous
import functools

import jax
import jax.numpy as jnp
from jax import lax
from jax.experimental import pallas as pl
from jax.experimental.pallas import tpu as pltpu
from jax.experimental.pallas import tpu_sc as plsc

NUM_BINS = 5
BATCH = 16384

_HALF_LOG_2PI = 0.9189385332046727
_CF_ITERS = 8
_FPMIN = 1e-30


_NC = 2
_NS = 16
_NW = _NC * _NS
_BPW = BATCH // _NW


def _sc_gather_body(uid_hbm, iid_hbm, u0, u1, u2, u3, u4, i0, i1, i2, i3, i4,
                    out_hbm, uidx_v, iidx_v, urows_v, irows_v, sem_u, sem_i):
    ut_bins = (u0, u1, u2, u3, u4)
    it_bins = (i0, i1, i2, i3, i4)
    wid = lax.axis_index("s") * _NC + lax.axis_index("c")
    base = wid * _BPW
    pltpu.sync_copy(uid_hbm.at[pl.ds(base, _BPW)], uidx_v)
    pltpu.sync_copy(iid_hbm.at[pl.ds(base, _BPW)], iidx_v)
    cps = []
    for k in range(NUM_BINS):
        cps.append(pltpu.async_copy(ut_bins[k].at[uidx_v],
                                    urows_v.at[pl.ds(k * _BPW, _BPW)], sem_u))
        cps.append(pltpu.async_copy(it_bins[k].at[iidx_v],
                                    irows_v.at[pl.ds(k * _BPW, _BPW)], sem_i))
    for cp in cps:
        cp.wait()

    def add_chunk(j, _):
        sl = pl.ds(j * 16, 16)
        urows_v[sl] = urows_v[sl] + irows_v[sl]
        return _
    lax.fori_loop(0, NUM_BINS * _BPW // 16, add_chunk, 0)
    for k in range(NUM_BINS):
        pltpu.sync_copy(urows_v.at[pl.ds(k * _BPW, _BPW)],
                        out_hbm.at[k, pl.ds(base, _BPW)])


def _sc_gather(uid, iid, u_bins, i_bins):
    mesh = plsc.VectorSubcoreMesh(core_axis_name="c", subcore_axis_name="s")
    f = pl.kernel(
        _sc_gather_body,
        mesh=mesh,
        compiler_params=pltpu.CompilerParams(use_tc_tiling_on_sc=False),
        out_type=jax.ShapeDtypeStruct((NUM_BINS, BATCH), jnp.float32),
        scratch_types=[
            pltpu.VMEM((_BPW,), jnp.int32),
            pltpu.VMEM((_BPW,), jnp.int32),
            pltpu.VMEM((NUM_BINS * _BPW,), jnp.float32),
            pltpu.VMEM((NUM_BINS * _BPW,), jnp.float32),
            pltpu.SemaphoreType.DMA,
            pltpu.SemaphoreType.DMA,
        ],
    )
    return f(uid, iid, *u_bins, *i_bins)


_NROWS = 1000000


_DETILE_BLK = 131072


def _tc_detile_body(u_ref, i_ref, *out_refs):
    for k in range(NUM_BINS):
        out_refs[k][...] = u_ref[k, :]
        out_refs[NUM_BINS + k][...] = i_ref[k, :]


def _tc_detile(ut_t, it_t):
    nblk = pl.cdiv(_NROWS, _DETILE_BLK)
    spec_in = pl.BlockSpec((NUM_BINS, _DETILE_BLK), lambda j: (0, j))
    spec_out = pl.BlockSpec((_DETILE_BLK,), lambda j: (j,))
    return pl.pallas_call(
        _tc_detile_body,
        grid=(nblk,),
        in_specs=[spec_in, spec_in],
        out_specs=[spec_out] * (2 * NUM_BINS),
        out_shape=tuple(jax.ShapeDtypeStruct((_NROWS,), jnp.float32)
                        for _ in range(2 * NUM_BINS)),
    )(ut_t, it_t)


def _lgamma_small(x):
    z = x + 6.0
    shift = (jnp.log(x) + jnp.log(x + 1.0) + jnp.log(x + 2.0)
             + jnp.log(x + 3.0) + jnp.log(x + 4.0) + jnp.log(x + 5.0))
    zi = 1.0 / z
    zi2 = zi * zi
    st = (z - 0.5) * jnp.log(z) - z + _HALF_LOG_2PI + zi * (
        1.0 / 12.0 - zi2 * (1.0 / 360.0))
    return st - shift


def _betainc_cf(a, b, x):
    thresh = (a + 1.0) / (a + b + 2.0)
    swap = x > thresh
    aa = jnp.where(swap, b, a)
    bb = jnp.where(swap, a, b)
    xx = jnp.where(swap, 1.0 - x, x)
    xx = jnp.clip(xx, _FPMIN, 1.0)
    lnpre = (aa * jnp.log(xx) + bb * jnp.log(1.0 - xx)
             + _lgamma_small(aa + bb) - _lgamma_small(aa) - _lgamma_small(bb))
    front = jnp.exp(lnpre) / aa
    qab = aa + bb
    qap = aa + 1.0
    qam = aa - 1.0
    c = jnp.ones_like(xx)
    d = 1.0 - qab * xx / qap
    d = jnp.where(jnp.abs(d) < _FPMIN, _FPMIN, d)
    d = 1.0 / d
    h = d
    for m in range(1, _CF_ITERS + 1):
        m2 = 2.0 * m
        num = m * (bb - m) * xx / ((qam + m2) * (aa + m2))
        d = 1.0 + num * d
        d = jnp.where(jnp.abs(d) < _FPMIN, _FPMIN, d)
        c = 1.0 + num / c
        c = jnp.where(jnp.abs(c) < _FPMIN, _FPMIN, c)
        d = 1.0 / d
        h = h * d * c
        num = -(aa + m) * (qab + m) * xx / ((aa + m2) * (qap + m2))
        d = 1.0 + num * d
        d = jnp.where(jnp.abs(d) < _FPMIN, _FPMIN, d)
        c = 1.0 + num / c
        c = jnp.where(jnp.abs(c) < _FPMIN, _FPMIN, c)
        d = 1.0 / d
        h = h * d * c
    res = front * h
    return jnp.where(swap, 1.0 - res, res)


def _tc_body(s_ref, a_ref, b_ref, mass_ref, edges_ref):
    e = jnp.exp(s_ref[...])
    r0 = e[0:1, :]
    r1 = r0 + e[1:2, :]
    r2 = r1 + e[2:3, :]
    r3 = r2 + e[3:4, :]
    tot = r3 + e[4:5, :]
    inv = 1.0 / tot
    e0 = r0 * inv
    e1 = r1 * inv
    e2 = r2 * inv
    e3 = r3 * inv
    ones = jnp.ones_like(e0)
    edges_ref[...] = jnp.concatenate([e0, e1, e2, e3, ones], axis=0)
    x4 = jnp.concatenate([e0, e1, e2, e3], axis=0)
    a4 = jnp.broadcast_to(a_ref[...], x4.shape)
    b4 = jnp.broadcast_to(b_ref[...], x4.shape)
    cdf = _betainc_cf(a4, b4, x4)
    c0 = cdf[0:1, :]
    c1 = cdf[1:2, :]
    c2 = cdf[2:3, :]
    c3 = cdf[3:4, :]
    mass_ref[...] = jnp.concatenate(
        [c0, c1 - c0, c2 - c1, c3 - c2, ones - c3], axis=0)


def _tc_math(s_t, a_t, b_t, block_l=2048):
    nblk = BATCH // block_l
    spec5 = pl.BlockSpec((NUM_BINS, block_l), lambda j: (0, j))
    spec1 = pl.BlockSpec((1, block_l), lambda j: (0, j))
    return pl.pallas_call(
        _tc_body,
        grid=(nblk,),
        in_specs=[spec5, spec1, spec1],
        out_specs=[spec5, spec5],
        out_shape=(
            jax.ShapeDtypeStruct((NUM_BINS, BATCH), jnp.float32),
            jax.ShapeDtypeStruct((NUM_BINS, BATCH), jnp.float32),
        ),
    )(s_t, a_t, b_t)


def kernel(uid, iid, alpha, beta, uid_table, iid_table):
    ut_t = uid_table.T
    it_t = iid_table.T
    bins = _tc_detile(ut_t, it_t)
    u_bins = bins[:NUM_BINS]
    i_bins = bins[NUM_BINS:]
    s_t = _sc_gather(uid.astype(jnp.int32), iid.astype(jnp.int32),
                     u_bins, i_bins)
    a_t = alpha.reshape(1, BATCH)
    b_t = beta.reshape(1, BATCH)
    mass_t, edges_t = _tc_math(s_t, a_t, b_t)
    return (mass_t.T, edges_t.T)


# speedup vs baseline: 11.7197x; 1.0245x over previous
import functools

import jax
import jax.numpy as jnp
from jax import lax
from jax.experimental import pallas as pl
from jax.experimental.pallas import tpu as pltpu
from jax.experimental.pallas import tpu_sc as plsc

NUM_BINS = 5
BATCH = 16384

_HALF_LOG_2PI = 0.9189385332046727
_CF_ITERS = 6
_FPMIN = 1e-30


_NC = 2
_NS = 16
_NW = _NC * _NS
_BPW = BATCH // _NW


def _sc_gather_body(uid_hbm, iid_hbm, u0, u1, u2, u3, u4, i0, i1, i2, i3, i4,
                    out_hbm, uidx_v, iidx_v, urows_v, irows_v, sem_u, sem_i):
    ut_bins = (u0, u1, u2, u3, u4)
    it_bins = (i0, i1, i2, i3, i4)
    wid = lax.axis_index("s") * _NC + lax.axis_index("c")
    base = wid * _BPW
    pltpu.sync_copy(uid_hbm.at[pl.ds(base, _BPW)], uidx_v)
    pltpu.sync_copy(iid_hbm.at[pl.ds(base, _BPW)], iidx_v)
    cps = []
    for k in range(NUM_BINS):
        cps.append(pltpu.async_copy(ut_bins[k].at[uidx_v],
                                    urows_v.at[pl.ds(k * _BPW, _BPW)], sem_u))
        cps.append(pltpu.async_copy(it_bins[k].at[iidx_v],
                                    irows_v.at[pl.ds(k * _BPW, _BPW)], sem_i))
    for cp in cps:
        cp.wait()

    def add_chunk(j, _):
        sl = pl.ds(j * 16, 16)
        urows_v[sl] = urows_v[sl] + irows_v[sl]
        return _
    lax.fori_loop(0, NUM_BINS * _BPW // 16, add_chunk, 0)
    for k in range(NUM_BINS):
        pltpu.sync_copy(urows_v.at[pl.ds(k * _BPW, _BPW)],
                        out_hbm.at[k, pl.ds(base, _BPW)])


def _sc_gather(uid, iid, u_bins, i_bins):
    mesh = plsc.VectorSubcoreMesh(core_axis_name="c", subcore_axis_name="s")
    f = pl.kernel(
        _sc_gather_body,
        mesh=mesh,
        compiler_params=pltpu.CompilerParams(use_tc_tiling_on_sc=False),
        out_type=jax.ShapeDtypeStruct((NUM_BINS, BATCH), jnp.float32),
        scratch_types=[
            pltpu.VMEM((_BPW,), jnp.int32),
            pltpu.VMEM((_BPW,), jnp.int32),
            pltpu.VMEM((NUM_BINS * _BPW,), jnp.float32),
            pltpu.VMEM((NUM_BINS * _BPW,), jnp.float32),
            pltpu.SemaphoreType.DMA,
            pltpu.SemaphoreType.DMA,
        ],
    )
    return f(uid, iid, *u_bins, *i_bins)


_NROWS = 1000000


_DETILE_BLK = 131072


def _tc_detile_body(u_ref, i_ref, *out_refs):
    for k in range(NUM_BINS):
        out_refs[k][...] = u_ref[k, :]
        out_refs[NUM_BINS + k][...] = i_ref[k, :]


def _tc_detile(ut_t, it_t):
    nblk = pl.cdiv(_NROWS, _DETILE_BLK)
    spec_in = pl.BlockSpec((NUM_BINS, _DETILE_BLK), lambda j: (0, j))
    spec_out = pl.BlockSpec((_DETILE_BLK,), lambda j: (j,))
    return pl.pallas_call(
        _tc_detile_body,
        grid=(nblk,),
        in_specs=[spec_in, spec_in],
        out_specs=[spec_out] * (2 * NUM_BINS),
        out_shape=tuple(jax.ShapeDtypeStruct((_NROWS,), jnp.float32)
                        for _ in range(2 * NUM_BINS)),
    )(ut_t, it_t)


def _lgamma_small(x):
    z = x + 6.0
    shift = (jnp.log(x) + jnp.log(x + 1.0) + jnp.log(x + 2.0)
             + jnp.log(x + 3.0) + jnp.log(x + 4.0) + jnp.log(x + 5.0))
    zi = 1.0 / z
    zi2 = zi * zi
    st = (z - 0.5) * jnp.log(z) - z + _HALF_LOG_2PI + zi * (
        1.0 / 12.0 - zi2 * (1.0 / 360.0))
    return st - shift


def _betainc_cf(a, b, x):
    a = jnp.maximum(a, 1e-12)
    b = jnp.maximum(b, 1e-12)
    thresh = (a + 1.0) / (a + b + 2.0)
    swap = x > thresh
    aa = jnp.where(swap, b, a)
    bb = jnp.where(swap, a, b)
    xx = jnp.where(swap, 1.0 - x, x)
    xx = jnp.clip(xx, _FPMIN, 1.0)
    lnpre = (aa * jnp.log(xx) + bb * jnp.log(1.0 - xx)
             + _lgamma_small(aa + bb) - _lgamma_small(aa) - _lgamma_small(bb))
    front = jnp.exp(lnpre) / aa
    qab = aa + bb
    qap = aa + 1.0
    qam = aa - 1.0
    c = jnp.ones_like(xx)
    d = 1.0 - qab * xx / qap
    d = jnp.where(jnp.abs(d) < _FPMIN, _FPMIN, d)
    d = 1.0 / d
    h = d
    for m in range(1, _CF_ITERS + 1):
        m2 = 2.0 * m
        num = m * (bb - m) * xx / ((qam + m2) * (aa + m2))
        d = 1.0 + num * d
        d = jnp.where(jnp.abs(d) < _FPMIN, _FPMIN, d)
        c = 1.0 + num / c
        c = jnp.where(jnp.abs(c) < _FPMIN, _FPMIN, c)
        d = 1.0 / d
        h = h * d * c
        num = -(aa + m) * (qab + m) * xx / ((aa + m2) * (qap + m2))
        d = 1.0 + num * d
        d = jnp.where(jnp.abs(d) < _FPMIN, _FPMIN, d)
        c = 1.0 + num / c
        c = jnp.where(jnp.abs(c) < _FPMIN, _FPMIN, c)
        d = 1.0 / d
        h = h * d * c
    res = front * h
    return jnp.where(swap, 1.0 - res, res)


def _tc_body(s_ref, a_ref, b_ref, mass_ref, edges_ref):
    e = jnp.exp(s_ref[...])
    r0 = e[0:1, :]
    r1 = r0 + e[1:2, :]
    r2 = r1 + e[2:3, :]
    r3 = r2 + e[3:4, :]
    tot = r3 + e[4:5, :]
    inv = 1.0 / tot
    e0 = r0 * inv
    e1 = r1 * inv
    e2 = r2 * inv
    e3 = r3 * inv
    ones = jnp.ones_like(e0)
    edges_ref[...] = jnp.concatenate([e0, e1, e2, e3, ones], axis=0)
    x4 = jnp.concatenate([e0, e1, e2, e3], axis=0)
    a4 = jnp.broadcast_to(a_ref[...], x4.shape)
    b4 = jnp.broadcast_to(b_ref[...], x4.shape)
    cdf = _betainc_cf(a4, b4, x4)
    c0 = cdf[0:1, :]
    c1 = cdf[1:2, :]
    c2 = cdf[2:3, :]
    c3 = cdf[3:4, :]
    mass_ref[...] = jnp.concatenate(
        [c0, c1 - c0, c2 - c1, c3 - c2, ones - c3], axis=0)


def _tc_math(s_t, a_t, b_t, block_l=2048):
    nblk = BATCH // block_l
    spec5 = pl.BlockSpec((NUM_BINS, block_l), lambda j: (0, j))
    spec1 = pl.BlockSpec((1, block_l), lambda j: (0, j))
    return pl.pallas_call(
        _tc_body,
        grid=(nblk,),
        in_specs=[spec5, spec1, spec1],
        out_specs=[spec5, spec5],
        out_shape=(
            jax.ShapeDtypeStruct((NUM_BINS, BATCH), jnp.float32),
            jax.ShapeDtypeStruct((NUM_BINS, BATCH), jnp.float32),
        ),
    )(s_t, a_t, b_t)


def kernel(uid, iid, alpha, beta, uid_table, iid_table):
    ut_t = uid_table.T
    it_t = iid_table.T
    bins = _tc_detile(ut_t, it_t)
    u_bins = bins[:NUM_BINS]
    i_bins = bins[NUM_BINS:]
    s_t = _sc_gather(uid.astype(jnp.int32), iid.astype(jnp.int32),
                     u_bins, i_bins)
    a_t = alpha.reshape(1, BATCH)
    b_t = beta.reshape(1, BATCH)
    mass_t, edges_t = _tc_math(s_t, a_t, b_t)
    return (mass_t.T, edges_t.T)


# speedup vs baseline: 12.0856x; 1.0312x over previous
import functools

import jax
import jax.numpy as jnp
from jax import lax
from jax.experimental import pallas as pl
from jax.experimental.pallas import tpu as pltpu
from jax.experimental.pallas import tpu_sc as plsc

NUM_BINS = 5
BATCH = 16384

_HALF_LOG_2PI = 0.9189385332046727
_CF_ITERS = 4
_FPMIN = 1e-30


_NC = 2
_NS = 16
_NW = _NC * _NS
_BPW = BATCH // _NW


def _sc_gather_body(uid_hbm, iid_hbm, u0, u1, u2, u3, u4, i0, i1, i2, i3, i4,
                    out_hbm, uidx_v, iidx_v, urows_v, irows_v, sem_u, sem_i):
    ut_bins = (u0, u1, u2, u3, u4)
    it_bins = (i0, i1, i2, i3, i4)
    wid = lax.axis_index("s") * _NC + lax.axis_index("c")
    base = wid * _BPW
    pltpu.sync_copy(uid_hbm.at[pl.ds(base, _BPW)], uidx_v)
    pltpu.sync_copy(iid_hbm.at[pl.ds(base, _BPW)], iidx_v)
    cps = []
    for k in range(NUM_BINS):
        cps.append(pltpu.async_copy(ut_bins[k].at[uidx_v],
                                    urows_v.at[pl.ds(k * _BPW, _BPW)], sem_u))
        cps.append(pltpu.async_copy(it_bins[k].at[iidx_v],
                                    irows_v.at[pl.ds(k * _BPW, _BPW)], sem_i))
    for cp in cps:
        cp.wait()

    def add_chunk(j, _):
        sl = pl.ds(j * 16, 16)
        urows_v[sl] = urows_v[sl] + irows_v[sl]
        return _
    lax.fori_loop(0, NUM_BINS * _BPW // 16, add_chunk, 0)
    for k in range(NUM_BINS):
        pltpu.sync_copy(urows_v.at[pl.ds(k * _BPW, _BPW)],
                        out_hbm.at[k, pl.ds(base, _BPW)])


def _sc_gather(uid, iid, u_bins, i_bins):
    mesh = plsc.VectorSubcoreMesh(core_axis_name="c", subcore_axis_name="s")
    f = pl.kernel(
        _sc_gather_body,
        mesh=mesh,
        compiler_params=pltpu.CompilerParams(use_tc_tiling_on_sc=False),
        out_type=jax.ShapeDtypeStruct((NUM_BINS, BATCH), jnp.float32),
        scratch_types=[
            pltpu.VMEM((_BPW,), jnp.int32),
            pltpu.VMEM((_BPW,), jnp.int32),
            pltpu.VMEM((NUM_BINS * _BPW,), jnp.float32),
            pltpu.VMEM((NUM_BINS * _BPW,), jnp.float32),
            pltpu.SemaphoreType.DMA,
            pltpu.SemaphoreType.DMA,
        ],
    )
    return f(uid, iid, *u_bins, *i_bins)


_NROWS = 1000000


_DETILE_BLK = 262144


def _tc_detile_body(u_ref, i_ref, *out_refs):
    for k in range(NUM_BINS):
        out_refs[k][...] = u_ref[k, :]
        out_refs[NUM_BINS + k][...] = i_ref[k, :]


def _tc_detile(ut_t, it_t):
    nblk = pl.cdiv(_NROWS, _DETILE_BLK)
    spec_in = pl.BlockSpec((NUM_BINS, _DETILE_BLK), lambda j: (0, j))
    spec_out = pl.BlockSpec((_DETILE_BLK,), lambda j: (j,))
    return pl.pallas_call(
        _tc_detile_body,
        grid=(nblk,),
        in_specs=[spec_in, spec_in],
        out_specs=[spec_out] * (2 * NUM_BINS),
        out_shape=tuple(jax.ShapeDtypeStruct((_NROWS,), jnp.float32)
                        for _ in range(2 * NUM_BINS)),
    )(ut_t, it_t)


def _lgamma_small(x):
    z = x + 6.0
    shift = (jnp.log(x) + jnp.log(x + 1.0) + jnp.log(x + 2.0)
             + jnp.log(x + 3.0) + jnp.log(x + 4.0) + jnp.log(x + 5.0))
    zi = 1.0 / z
    zi2 = zi * zi
    st = (z - 0.5) * jnp.log(z) - z + _HALF_LOG_2PI + zi * (
        1.0 / 12.0 - zi2 * (1.0 / 360.0))
    return st - shift


def _betainc_cf(a, b, x):
    a = jnp.maximum(a, 1e-12)
    b = jnp.maximum(b, 1e-12)
    thresh = (a + 1.0) / (a + b + 2.0)
    swap = x > thresh
    aa = jnp.where(swap, b, a)
    bb = jnp.where(swap, a, b)
    xx = jnp.where(swap, 1.0 - x, x)
    xx = jnp.clip(xx, _FPMIN, 1.0)
    lnpre = (aa * jnp.log(xx) + bb * jnp.log(1.0 - xx)
             + _lgamma_small(aa + bb) - _lgamma_small(aa) - _lgamma_small(bb))
    front = jnp.exp(lnpre) / aa
    qab = aa + bb
    qap = aa + 1.0
    qam = aa - 1.0
    c = jnp.ones_like(xx)
    d = 1.0 - qab * xx / qap
    d = jnp.where(jnp.abs(d) < _FPMIN, _FPMIN, d)
    d = 1.0 / d
    h = d
    for m in range(1, _CF_ITERS + 1):
        m2 = 2.0 * m
        num = m * (bb - m) * xx / ((qam + m2) * (aa + m2))
        d = 1.0 + num * d
        d = jnp.where(jnp.abs(d) < _FPMIN, _FPMIN, d)
        c = 1.0 + num / c
        c = jnp.where(jnp.abs(c) < _FPMIN, _FPMIN, c)
        d = 1.0 / d
        h = h * d * c
        num = -(aa + m) * (qab + m) * xx / ((aa + m2) * (qap + m2))
        d = 1.0 + num * d
        d = jnp.where(jnp.abs(d) < _FPMIN, _FPMIN, d)
        c = 1.0 + num / c
        c = jnp.where(jnp.abs(c) < _FPMIN, _FPMIN, c)
        d = 1.0 / d
        h = h * d * c
    res = front * h
    return jnp.where(swap, 1.0 - res, res)


def _tc_body(s_ref, a_ref, b_ref, mass_ref, edges_ref):
    e = jnp.exp(s_ref[...])
    r0 = e[0:1, :]
    r1 = r0 + e[1:2, :]
    r2 = r1 + e[2:3, :]
    r3 = r2 + e[3:4, :]
    tot = r3 + e[4:5, :]
    inv = 1.0 / tot
    e0 = r0 * inv
    e1 = r1 * inv
    e2 = r2 * inv
    e3 = r3 * inv
    ones = jnp.ones_like(e0)
    edges_ref[...] = jnp.concatenate([e0, e1, e2, e3, ones], axis=0)
    x4 = jnp.concatenate([e0, e1, e2, e3], axis=0)
    a4 = jnp.broadcast_to(a_ref[...], x4.shape)
    b4 = jnp.broadcast_to(b_ref[...], x4.shape)
    cdf = _betainc_cf(a4, b4, x4)
    c0 = cdf[0:1, :]
    c1 = cdf[1:2, :]
    c2 = cdf[2:3, :]
    c3 = cdf[3:4, :]
    mass_ref[...] = jnp.concatenate(
        [c0, c1 - c0, c2 - c1, c3 - c2, ones - c3], axis=0)


def _tc_math(s_t, a_t, b_t, block_l=4096):
    nblk = BATCH // block_l
    spec5 = pl.BlockSpec((NUM_BINS, block_l), lambda j: (0, j))
    spec1 = pl.BlockSpec((1, block_l), lambda j: (0, j))
    return pl.pallas_call(
        _tc_body,
        grid=(nblk,),
        in_specs=[spec5, spec1, spec1],
        out_specs=[spec5, spec5],
        out_shape=(
            jax.ShapeDtypeStruct((NUM_BINS, BATCH), jnp.float32),
            jax.ShapeDtypeStruct((NUM_BINS, BATCH), jnp.float32),
        ),
    )(s_t, a_t, b_t)


def kernel(uid, iid, alpha, beta, uid_table, iid_table):
    ut_t = uid_table.T
    it_t = iid_table.T
    bins = _tc_detile(ut_t, it_t)
    u_bins = bins[:NUM_BINS]
    i_bins = bins[NUM_BINS:]
    s_t = _sc_gather(uid.astype(jnp.int32), iid.astype(jnp.int32),
                     u_bins, i_bins)
    a_t = alpha.reshape(1, BATCH)
    b_t = beta.reshape(1, BATCH)
    mass_t, edges_t = _tc_math(s_t, a_t, b_t)
    return (mass_t.T, edges_t.T)


# speedup vs baseline: 12.1162x; 1.0025x over previous
import functools

import jax
import jax.numpy as jnp
from jax import lax
from jax.experimental import pallas as pl
from jax.experimental.pallas import tpu as pltpu
from jax.experimental.pallas import tpu_sc as plsc

NUM_BINS = 5
BATCH = 16384

_HALF_LOG_2PI = 0.9189385332046727
_CF_ITERS = 4
_FPMIN = 1e-30


_NC = 2
_NS = 16
_NW = _NC * _NS
_BPW = BATCH // _NW


def _sc_gather_body(idx_hbm, b0, b1, b2, b3, b4,
                    out_hbm, idx_v, rows_v, sem):
    tbins = (b0, b1, b2, b3, b4)
    wid = lax.axis_index("s") * _NC + lax.axis_index("c")
    base = wid * _BPW
    pltpu.sync_copy(idx_hbm.at[pl.ds(base, _BPW)], idx_v)
    cps = []
    for k in range(NUM_BINS):
        cps.append(pltpu.async_copy(tbins[k].at[idx_v],
                                    rows_v.at[pl.ds(k * _BPW, _BPW)], sem))
    for cp in cps:
        cp.wait()
    for k in range(NUM_BINS):
        pltpu.sync_copy(rows_v.at[pl.ds(k * _BPW, _BPW)],
                        out_hbm.at[k, pl.ds(base, _BPW)])


def _sc_gather(idx, t_bins):
    mesh = plsc.VectorSubcoreMesh(core_axis_name="c", subcore_axis_name="s")
    f = pl.kernel(
        _sc_gather_body,
        mesh=mesh,
        compiler_params=pltpu.CompilerParams(use_tc_tiling_on_sc=False),
        out_type=jax.ShapeDtypeStruct((NUM_BINS, BATCH), jnp.float32),
        scratch_types=[
            pltpu.VMEM((_BPW,), jnp.int32),
            pltpu.VMEM((NUM_BINS * _BPW,), jnp.float32),
            pltpu.SemaphoreType.DMA,
        ],
    )
    return f(idx, *t_bins)


_NROWS = 1000000


_DETILE_BLK = 262144


def _tc_detile_body(t_ref, *out_refs):
    for k in range(NUM_BINS):
        out_refs[k][...] = t_ref[k, :]


def _tc_detile(t_t):
    nblk = pl.cdiv(_NROWS, _DETILE_BLK)
    spec_in = pl.BlockSpec((NUM_BINS, _DETILE_BLK), lambda j: (0, j))
    spec_out = pl.BlockSpec((_DETILE_BLK,), lambda j: (j,))
    return pl.pallas_call(
        _tc_detile_body,
        grid=(nblk,),
        in_specs=[spec_in],
        out_specs=[spec_out] * NUM_BINS,
        out_shape=tuple(jax.ShapeDtypeStruct((_NROWS,), jnp.float32)
                        for _ in range(NUM_BINS)),
    )(t_t)


def _lgamma_small(x):
    z = x + 6.0
    shift = (jnp.log(x) + jnp.log(x + 1.0) + jnp.log(x + 2.0)
             + jnp.log(x + 3.0) + jnp.log(x + 4.0) + jnp.log(x + 5.0))
    zi = 1.0 / z
    zi2 = zi * zi
    st = (z - 0.5) * jnp.log(z) - z + _HALF_LOG_2PI + zi * (
        1.0 / 12.0 - zi2 * (1.0 / 360.0))
    return st - shift


def _betainc_cf(a, b, x):
    a = jnp.maximum(a, 1e-12)
    b = jnp.maximum(b, 1e-12)
    thresh = (a + 1.0) / (a + b + 2.0)
    swap = x > thresh
    aa = jnp.where(swap, b, a)
    bb = jnp.where(swap, a, b)
    xx = jnp.where(swap, 1.0 - x, x)
    xx = jnp.clip(xx, _FPMIN, 1.0)
    lnpre = (aa * jnp.log(xx) + bb * jnp.log(1.0 - xx)
             + _lgamma_small(aa + bb) - _lgamma_small(aa) - _lgamma_small(bb))
    front = jnp.exp(lnpre) / aa
    qab = aa + bb
    qap = aa + 1.0
    qam = aa - 1.0
    c = jnp.ones_like(xx)
    d = 1.0 - qab * xx / qap
    d = jnp.where(jnp.abs(d) < _FPMIN, _FPMIN, d)
    d = 1.0 / d
    h = d
    for m in range(1, _CF_ITERS + 1):
        m2 = 2.0 * m
        num = m * (bb - m) * xx / ((qam + m2) * (aa + m2))
        d = 1.0 + num * d
        d = jnp.where(jnp.abs(d) < _FPMIN, _FPMIN, d)
        c = 1.0 + num / c
        c = jnp.where(jnp.abs(c) < _FPMIN, _FPMIN, c)
        d = 1.0 / d
        h = h * d * c
        num = -(aa + m) * (qab + m) * xx / ((aa + m2) * (qap + m2))
        d = 1.0 + num * d
        d = jnp.where(jnp.abs(d) < _FPMIN, _FPMIN, d)
        c = 1.0 + num / c
        c = jnp.where(jnp.abs(c) < _FPMIN, _FPMIN, c)
        d = 1.0 / d
        h = h * d * c
    res = front * h
    return jnp.where(swap, 1.0 - res, res)


def _tc_body(su_ref, si_ref, a_ref, b_ref, mass_ref, edges_ref):
    e = jnp.exp(su_ref[...] + si_ref[...])
    r0 = e[0:1, :]
    r1 = r0 + e[1:2, :]
    r2 = r1 + e[2:3, :]
    r3 = r2 + e[3:4, :]
    tot = r3 + e[4:5, :]
    inv = 1.0 / tot
    e0 = r0 * inv
    e1 = r1 * inv
    e2 = r2 * inv
    e3 = r3 * inv
    ones = jnp.ones_like(e0)
    edges_ref[...] = jnp.concatenate([e0, e1, e2, e3, ones], axis=0)
    x4 = jnp.concatenate([e0, e1, e2, e3], axis=0)
    a4 = jnp.broadcast_to(a_ref[...], x4.shape)
    b4 = jnp.broadcast_to(b_ref[...], x4.shape)
    cdf = _betainc_cf(a4, b4, x4)
    c0 = cdf[0:1, :]
    c1 = cdf[1:2, :]
    c2 = cdf[2:3, :]
    c3 = cdf[3:4, :]
    mass_ref[...] = jnp.concatenate(
        [c0, c1 - c0, c2 - c1, c3 - c2, ones - c3], axis=0)


def _tc_math(su_t, si_t, a_t, b_t, block_l=4096):
    nblk = BATCH // block_l
    spec5 = pl.BlockSpec((NUM_BINS, block_l), lambda j: (0, j))
    spec1 = pl.BlockSpec((1, block_l), lambda j: (0, j))
    return pl.pallas_call(
        _tc_body,
        grid=(nblk,),
        in_specs=[spec5, spec5, spec1, spec1],
        out_specs=[spec5, spec5],
        out_shape=(
            jax.ShapeDtypeStruct((NUM_BINS, BATCH), jnp.float32),
            jax.ShapeDtypeStruct((NUM_BINS, BATCH), jnp.float32),
        ),
    )(su_t, si_t, a_t, b_t)


def kernel(uid, iid, alpha, beta, uid_table, iid_table):
    u_bins = _tc_detile(uid_table.T)
    su_t = _sc_gather(uid.astype(jnp.int32), u_bins)
    i_bins = _tc_detile(iid_table.T)
    si_t = _sc_gather(iid.astype(jnp.int32), i_bins)
    a_t = alpha.reshape(1, BATCH)
    b_t = beta.reshape(1, BATCH)
    mass_t, edges_t = _tc_math(su_t, si_t, a_t, b_t)
    return (mass_t.T, edges_t.T)
